# Initial kernel scaffold; baseline (speedup 1.0000x reference)
import jax
import jax.numpy as jnp
from jax.experimental import pallas as pl


def kernel(x, edge_index, W1, b1, W2, b2):
    raise NotImplementedError("write your pallas kernel here")


# speedup vs baseline: 18.6518x; 18.6518x over previous
import functools

import jax
import jax.numpy as jnp
from jax import lax
from jax.experimental import pallas as pl
from jax.experimental.pallas import tpu as pltpu
from jax.experimental.pallas import tpu_sc as plsc

N = 10000
D = 128
E = 320000
NC, NS = 2, 16
NW = NC * NS
EPW = E // NW
K = 80
NCH = EPW // K
NP = 10240
SROWS = NP // NS
RB = 400
NRB = N // RB

_mesh = plsc.VectorSubcoreMesh(
    core_axis_name="c", subcore_axis_name="s", num_cores=NC, num_subcores=NS
)


def _deg_body(dst_hbm, out_hbm, dst_v, hist_v):
    c = lax.axis_index("c")
    s = lax.axis_index("s")
    wid = s * NC + c
    zeros16 = jnp.zeros((16,), jnp.float32)
    ones16 = jnp.ones((16,), jnp.float32)

    def zero_it(j, carry):
        hist_v[pl.ds(j * 16, 16)] = zeros16
        return carry

    lax.fori_loop(0, N // 16, zero_it, 0)
    pltpu.sync_copy(dst_hbm.at[pl.ds(wid * EPW, EPW)], dst_v)

    def hist_it(j, carry):
        idx = dst_v[pl.ds(j * 16, 16)]
        plsc.addupdate_scatter(hist_v, [idx], ones16)
        return carry

    lax.fori_loop(0, EPW // 16, hist_it, 0)
    pltpu.sync_copy(hist_v, out_hbm.at[wid])


_deg_call = functools.partial(
    pl.kernel,
    out_type=jax.ShapeDtypeStruct((NW, N), jnp.float32),
    mesh=_mesh,
    scratch_types=[
        pltpu.VMEM((EPW,), jnp.int32),
        pltpu.VMEM((N,), jnp.float32),
    ],
    compiler_params=pltpu.CompilerParams(needs_layout_passes=False),
    name="sc_degree",
)(_deg_body)


def _agg_body(g_hbm, src_hbm, dst_hbm, zer_hbm, out_hbm, srcv, dstv, rows,
              acc_sh, sem):
    c = lax.axis_index("c")
    s = lax.axis_index("s")
    wid = s * NC + c
    pltpu.sync_copy(zer_hbm, acc_sh.at[pl.ds(s * SROWS, SROWS)])
    pltpu.sync_copy(src_hbm.at[wid], srcv)
    pltpu.sync_copy(dst_hbm.at[wid], dstv)
    plsc.subcore_barrier()

    def chunk(i, carry):
        pltpu.async_copy(g_hbm.at[srcv.at[i]], rows, sem).wait()
        pltpu.sync_copy(rows, acc_sh.at[dstv.at[i]], add=True)
        return carry

    lax.fori_loop(0, NCH, chunk, 0)
    plsc.subcore_barrier()
    pltpu.sync_copy(
        acc_sh.at[pl.ds(s * SROWS, SROWS)],
        out_hbm.at[pl.ds(c * NP + s * SROWS, SROWS)],
    )


_agg_call = functools.partial(
    pl.kernel,
    out_type=jax.ShapeDtypeStruct((NC * NP, D), jnp.float32),
    mesh=_mesh,
    scratch_types=[
        pltpu.VMEM((NCH, K), jnp.int32),
        pltpu.VMEM((NCH, K), jnp.int32),
        pltpu.VMEM((K, D), jnp.float32),
        pltpu.VMEM_SHARED((NP, D), jnp.float32),
        pltpu.SemaphoreType.DMA,
    ],
    name="sc_aggregate",
)(_agg_body)


def _dinv(degt_ref):
    dsum = jnp.sum(degt_ref[...], axis=1, keepdims=True) + 1.0
    return lax.rsqrt(dsum)


def _mm1_body(x_ref, w_ref, degt_ref, o_ref):
    o_ref[...] = (
        jnp.dot(x_ref[...], w_ref[...], preferred_element_type=jnp.float32)
        * _dinv(degt_ref)
    )


def _mid_body(a0_ref, a1_ref, g_ref, degt_ref, b_ref, w_ref, o_ref):
    dinv = _dinv(degt_ref)
    z = (a0_ref[...] + a1_ref[...] + g_ref[...]) * dinv + b_ref[...]
    h = jnp.tanh(z)
    o_ref[...] = (
        jnp.dot(h, w_ref[...], preferred_element_type=jnp.float32) * dinv
    )


def _fin_body(a0_ref, a1_ref, g_ref, degt_ref, b_ref, o_ref):
    dinv = _dinv(degt_ref)
    z = (a0_ref[...] + a1_ref[...] + g_ref[...]) * dinv + b_ref[...]
    o_ref[...] = jnp.tanh(z)


_row = pl.BlockSpec((RB, D), lambda i: (i, 0))
_wspec = pl.BlockSpec((D, D), lambda i: (0, 0))
_dspec = pl.BlockSpec((RB, NW), lambda i: (i, 0))
_bspec = pl.BlockSpec((1, D), lambda i: (0, 0))
_oshape = jax.ShapeDtypeStruct((N, D), jnp.float32)

_mm1 = pl.pallas_call(
    _mm1_body, grid=(NRB,),
    in_specs=[_row, _wspec, _dspec], out_specs=_row, out_shape=_oshape,
)
_mid = pl.pallas_call(
    _mid_body, grid=(NRB,),
    in_specs=[_row, _row, _row, _dspec, _bspec, _wspec],
    out_specs=_row, out_shape=_oshape,
)
_fin = pl.pallas_call(
    _fin_body, grid=(NRB,),
    in_specs=[_row, _row, _row, _dspec, _bspec],
    out_specs=_row, out_shape=_oshape,
)


def kernel(x, edge_index, W1, b1, W2, b2):
    ei = edge_index.astype(jnp.int32)
    src = ei[0].reshape(NW, NCH, K)
    dst_flat = ei[1]
    dst = dst_flat.reshape(NW, NCH, K)
    zer = jnp.zeros((SROWS, D), jnp.float32)
    b1r = b1.reshape(1, D)
    b2r = b2.reshape(1, D)

    degp = _deg_call(dst_flat)
    degt = degp.T

    g1 = _mm1(x, W1, degt)
    acc1 = _agg_call(g1, src, dst, zer)
    g2 = _mid(acc1[:N], acc1[NP:NP + N], g1, degt, b1r, W2)
    acc2 = _agg_call(g2, src, dst, zer)
    return _fin(acc2[:N], acc2[NP:NP + N], g2, degt, b2r)
